# baseline (speedup 1.0000x reference)
import functools

import jax
import jax.numpy as jnp
import numpy as np
from jax import lax
from jax.experimental import pallas as pl
from jax.experimental.pallas import tpu as pltpu
from jax.experimental.pallas import tpu_sc as plsc

BATCH = 16384
HIST = 200
EMBED = 64
NCLS = 100
VOCAB = 1000000

NC = 2
NS = 16
NW = NC * NS
S_PER_W = BATCH // NW
SB = 64
NBLK = S_PER_W // SB
G = 8
NG = SB // G
CH0 = 104
CH1 = HIST - CH0
RU = 8
LANES = 16
PK = EMBED // (2 * LANES)

_PERM = np.concatenate([
    np.arange(0, 32, 2), np.arange(32, 64, 2),
    np.arange(1, 32, 2), np.arange(33, 64, 2)])

_mesh = plsc.VectorSubcoreMesh(core_axis_name="c", subcore_axis_name="s")


@functools.partial(
    pl.kernel,
    mesh=_mesh,
    out_type=jax.ShapeDtypeStruct((BATCH, EMBED), jnp.float32),
    scratch_types=[
        pltpu.VMEM((SB * HIST,), jnp.int32),
        pltpu.VMEM((2, G * HIST, 2 * LANES), jnp.int32),
        pltpu.VMEM((SB, EMBED), jnp.float32),
        pltpu.SemaphoreType.DMA,
        pltpu.SemaphoreType.DMA,
    ],
    compiler_params=pltpu.CompilerParams(use_tc_tiling_on_sc=False),
)
def _pool_kernel(x_hbm, table_hbm, pooled_hbm, idx_v, rows_v, pooled_v,
                 sem_a, sem_b):
    wid = lax.axis_index("s") * NC + lax.axis_index("c")
    base = wid * S_PER_W
    himask = jnp.int32(-65536)

    def fire_group(g, buf, sem):
        for j in range(G):
            s_local = g * G + j
            off = pl.multiple_of(s_local * HIST, 8)
            pltpu.async_copy(
                table_hbm.at[idx_v.at[pl.ds(off, CH0)]],
                rows_v.at[buf, pl.ds(j * HIST, CH0)], sem)
            off1 = pl.multiple_of(s_local * HIST + CH0, 8)
            pltpu.async_copy(
                table_hbm.at[idx_v.at[pl.ds(off1, CH1)]],
                rows_v.at[buf, pl.ds(j * HIST + CH0, CH1)], sem)

    def drain_group(buf, sem):
        pltpu.make_async_copy(
            table_hbm.at[pl.ds(0, G * HIST)], rows_v.at[buf], sem).wait()

    def accumulate(buf, g):
        for j in range(G):
            base_row = j * HIST
            zero = jnp.zeros((LANES,), jnp.float32)

            def body(r, acc):
                acc = list(acc)
                for rr in range(RU):
                    row = base_row + r * RU + rr
                    st = (rr % 2) * 4
                    for q in range(PK):
                        w = rows_v[buf, row, pl.ds(q * LANES, LANES)]
                        lo = lax.bitcast_convert_type(
                            lax.shift_left(w, 16), jnp.float32)
                        hi = lax.bitcast_convert_type(
                            jnp.bitwise_and(w, himask), jnp.float32)
                        acc[st + q] = acc[st + q] + lo
                        acc[st + 2 + q] = acc[st + 2 + q] + hi
                return tuple(acc)

            acc = lax.fori_loop(0, HIST // RU, body, (zero,) * 8)
            s_local = g * G + j
            for h in range(4):
                pooled_v[s_local, pl.ds(h * LANES, LANES)] = (
                    acc[h] + acc[4 + h])

    def block_body(blk, carry):
        row0 = base + blk * SB
        pltpu.sync_copy(x_hbm.at[pl.ds(row0 * HIST, SB * HIST)], idx_v)
        fire_group(0, 0, sem_a)

        def two_groups(p, c):
            g0 = 2 * p
            fire_group(g0 + 1, 1, sem_b)
            drain_group(0, sem_a)
            accumulate(0, g0)

            @pl.when(g0 + 2 < NG)
            def _():
                fire_group(g0 + 2, 0, sem_a)

            drain_group(1, sem_b)
            accumulate(1, g0 + 1)
            return c

        lax.fori_loop(0, NG // 2, two_groups, 0)
        pltpu.sync_copy(pooled_v, pooled_hbm.at[pl.ds(row0, SB)])
        return carry

    lax.fori_loop(0, NBLK, block_body, 0)


def _cls_body(p_ref, w_ref, b_ref, o_ref):
    o_ref[...] = lax.dot_general(
        p_ref[...], w_ref[...], (((1,), (1,)), ((), ())),
        preferred_element_type=jnp.float32) * (1.0 / HIST) + b_ref[...]


_BM = 2048


def kernel(x, table, W, b):
    x_flat = x.astype(jnp.int32).reshape(BATCH * HIST)
    packed = lax.bitcast_convert_type(
        table.astype(jnp.bfloat16).reshape(VOCAB, 2 * LANES, 2), jnp.int32)
    pooled = _pool_kernel(x_flat, packed)
    w_perm = W[:, _PERM]
    out = pl.pallas_call(
        _cls_body,
        grid=(BATCH // _BM,),
        in_specs=[
            pl.BlockSpec((_BM, EMBED), lambda i: (i, 0)),
            pl.BlockSpec((NCLS, EMBED), lambda i: (0, 0)),
            pl.BlockSpec((1, NCLS), lambda i: (0, 0)),
        ],
        out_specs=pl.BlockSpec((_BM, NCLS), lambda i: (i, 0)),
        out_shape=jax.ShapeDtypeStruct((BATCH, NCLS), jnp.float32),
    )(pooled, w_perm, b.reshape(1, NCLS))
    return out


# speedup vs baseline: 1.8080x; 1.8080x over previous
import functools

import jax
import jax.numpy as jnp
import numpy as np
from jax import lax
from jax.experimental import pallas as pl
from jax.experimental.pallas import tpu as pltpu
from jax.experimental.pallas import tpu_sc as plsc

BATCH = 16384
HIST = 200
EMBED = 64
NCLS = 100
VOCAB = 1000000

NC = 2
NS = 16
NW = NC * NS
S_PER_W = BATCH // NW
SB = 64
NBLK = S_PER_W // SB
G = 8
NG = SB // G
CH0 = 104
CH1 = HIST - CH0
RU = 8
LANES = 16
PK = EMBED // (2 * LANES)

_PERM = np.concatenate([
    np.arange(0, 32, 2), np.arange(32, 64, 2),
    np.arange(1, 32, 2), np.arange(33, 64, 2)])

_mesh = plsc.VectorSubcoreMesh(core_axis_name="c", subcore_axis_name="s")


@functools.partial(
    pl.kernel,
    mesh=_mesh,
    out_type=jax.ShapeDtypeStruct((BATCH, EMBED), jnp.float32),
    scratch_types=[
        pltpu.VMEM((SB * HIST,), jnp.int32),
        pltpu.VMEM((2, G * HIST, EMBED), jnp.bfloat16),
        pltpu.VMEM((SB, EMBED), jnp.float32),
        pltpu.SemaphoreType.DMA,
        pltpu.SemaphoreType.DMA,
    ],
    compiler_params=pltpu.CompilerParams(
        use_tc_tiling_on_sc=False, needs_layout_passes=False),
)
def _pool_kernel(x_hbm, table_hbm, pooled_hbm, idx_v, rows_v, pooled_v,
                 sem_a, sem_b):
    wid = lax.axis_index("s") * NC + lax.axis_index("c")
    base = wid * S_PER_W
    himask = jnp.int32(-65536)

    def fire_group(g, buf, sem):
        for j in range(G):
            s_local = g * G + j
            off = pl.multiple_of(s_local * HIST, 8)
            pltpu.async_copy(
                table_hbm.at[idx_v.at[pl.ds(off, CH0)]],
                rows_v.at[buf, pl.ds(j * HIST, CH0)], sem)
            off1 = pl.multiple_of(s_local * HIST + CH0, 8)
            pltpu.async_copy(
                table_hbm.at[idx_v.at[pl.ds(off1, CH1)]],
                rows_v.at[buf, pl.ds(j * HIST + CH0, CH1)], sem)

    def drain_group(buf, sem):
        pltpu.make_async_copy(
            table_hbm.at[pl.ds(0, G * HIST)], rows_v.at[buf], sem).wait()

    def accumulate(buf, g):
        for j in range(G):
            base_row = j * HIST
            zero = jnp.zeros((LANES,), jnp.float32)

            def body(r, acc):
                acc = list(acc)
                for rr in range(RU):
                    row = base_row + r * RU + rr
                    st = (rr % 2) * 4
                    for q in range(PK):
                        w16 = rows_v[buf, row, pl.ds(q * 2 * LANES, 2 * LANES)]
                        w = plsc.bitcast(w16, jnp.int32)
                        lo = lax.bitcast_convert_type(
                            lax.shift_left(w, 16), jnp.float32)
                        hi = lax.bitcast_convert_type(
                            jnp.bitwise_and(w, himask), jnp.float32)
                        acc[st + q] = acc[st + q] + lo
                        acc[st + 2 + q] = acc[st + 2 + q] + hi
                return tuple(acc)

            acc = lax.fori_loop(0, HIST // RU, body, (zero,) * 8)
            s_local = g * G + j
            for h in range(4):
                pooled_v[s_local, pl.ds(h * LANES, LANES)] = (
                    acc[h] + acc[4 + h])

    def block_body(blk, carry):
        row0 = base + blk * SB
        pltpu.sync_copy(x_hbm.at[pl.ds(row0 * HIST, SB * HIST)], idx_v)
        fire_group(0, 0, sem_a)

        def two_groups(p, c):
            g0 = 2 * p
            fire_group(g0 + 1, 1, sem_b)
            drain_group(0, sem_a)
            accumulate(0, g0)

            @pl.when(g0 + 2 < NG)
            def _():
                fire_group(g0 + 2, 0, sem_a)

            drain_group(1, sem_b)
            accumulate(1, g0 + 1)
            return c

        lax.fori_loop(0, NG // 2, two_groups, 0)
        pltpu.sync_copy(pooled_v, pooled_hbm.at[pl.ds(row0, SB)])
        return carry

    lax.fori_loop(0, NBLK, block_body, 0)


def _cls_body(p_ref, w_ref, b_ref, o_ref):
    o_ref[...] = lax.dot_general(
        p_ref[...], w_ref[...], (((1,), (1,)), ((), ())),
        preferred_element_type=jnp.float32) * (1.0 / HIST) + b_ref[...]


_BM = 2048


def kernel(x, table, W, b):
    x_flat = x.astype(jnp.int32).reshape(BATCH * HIST)
    tb16 = table.astype(jnp.bfloat16)
    pooled = _pool_kernel(x_flat, tb16)
    w_perm = W[:, _PERM]
    out = pl.pallas_call(
        _cls_body,
        grid=(BATCH // _BM,),
        in_specs=[
            pl.BlockSpec((_BM, EMBED), lambda i: (i, 0)),
            pl.BlockSpec((NCLS, EMBED), lambda i: (0, 0)),
            pl.BlockSpec((1, NCLS), lambda i: (0, 0)),
        ],
        out_specs=pl.BlockSpec((_BM, NCLS), lambda i: (i, 0)),
        out_shape=jax.ShapeDtypeStruct((BATCH, NCLS), jnp.float32),
    )(pooled, w_perm, b.reshape(1, NCLS))
    return out
